# Initial kernel scaffold; baseline (speedup 1.0000x reference)
import jax
import jax.numpy as jnp
from jax import lax
from jax.experimental import pallas as pl
from jax.experimental.pallas import tpu as pltpu
from jax.experimental.pallas import tpu_sc as plsc

N_MOLS = 10000
N_KEYS = 160000
D = 256
L = 16
NC = 2
NS = 16
NW = NC * NS
BLK = 128
NBLK_TOTAL = N_KEYS // BLK
BASE_BLKS = NBLK_TOTAL // NW
EXTRA = NBLK_TOTAL - BASE_BLKS * NW
MAXB = BASE_BLKS + 1
MAXK = MAXB * BLK
BASEK = BASE_BLKS * BLK
MROWSP = 640
INV_SQRT_D = 0.0625


def _qm_body(q_ref, wq_ref, wk_ref, out_ref):
    m = lax.dot_general(
        wq_ref[...], wk_ref[...], (((0,), (0,)), ((), ())),
        preferred_element_type=jnp.float32, precision=lax.Precision.HIGHEST)
    out_ref[...] = lax.dot_general(
        q_ref[...], m, (((1,), (0,)), ((), ())),
        preferred_element_type=jnp.float32, precision=lax.Precision.HIGHEST)


def _compute_qm(query, wq, wk):
    GB = 1250
    return pl.pallas_call(
        _qm_body,
        grid=(N_MOLS // GB,),
        in_specs=[
            pl.BlockSpec((GB, D), lambda i: (i, 0)),
            pl.BlockSpec((D, D), lambda i: (0, 0)),
            pl.BlockSpec((D, D), lambda i: (0, 0)),
        ],
        out_specs=pl.BlockSpec((GB, D), lambda i: (i, 0)),
        out_shape=jax.ShapeDtypeStruct((N_MOLS, D), jnp.float32),
    )(query, wq, wk)


def _worker_range(wid):
    nblk = BASE_BLKS + jnp.where(wid < EXTRA, 1, 0)
    kstart = (wid * BASE_BLKS + jnp.minimum(wid, EXTRA)) * BLK
    return nblk, kstart


def _pass1_body(qm_hbm, key_hbm, idx_hbm, scores_hbm, e_hbm, part_hbm,
                idx_v, keys_v, rows_v, bidx_v, sc_v, e_v, s_acc, e_acc,
                pz_v, midx_v, shared, sem_k, sem_g):
    cid = lax.axis_index("c")
    sid = lax.axis_index("s")
    wid = cid * NS + sid
    nblk, kstart = _worker_range(wid)
    nkeys = nblk * BLK

    zer = jnp.zeros((L,), jnp.float32)

    def zbody(i, _):
        s_acc[i, :] = zer
        e_acc[i, :] = zer
        pz_v[i, :] = zer
        return 0

    lax.fori_loop(0, MROWSP, zbody, 0)

    iota = lax.iota(jnp.int32, L)
    for t in range(5):
        for j8 in range(8):
            midx_v[t, pl.ds(j8 * 16, 16)] = iota + (t * 128 + j8 * 16)

    @pl.when(sid == 0)
    def _():
        pltpu.sync_copy(pz_v, shared)

    plsc.subcore_barrier()

    pltpu.sync_copy(idx_hbm.at[pl.ds(kstart, BASEK)], idx_v.at[pl.ds(8, BASEK)])

    @pl.when(nblk == MAXB)
    def _():
        pltpu.sync_copy(idx_hbm.at[pl.ds(kstart + BASEK, BLK)],
                        idx_v.at[pl.ds(8 + BASEK, BLK)])

    idx_v[7] = -1
    idx_v[8 + nkeys] = -2

    def block_body(b, cv):
        k0 = b * BLK
        for j8 in range(BLK // 16):
            bidx_v[pl.ds(j8 * 16, 16)] = idx_v[pl.ds(8 + k0 + j8 * 16, 16)]
        cp_k = pltpu.make_async_copy(
            key_hbm.at[pl.ds(kstart + k0, BLK)], keys_v, sem_k)
        cp_k.start()
        cp_g = pltpu.make_async_copy(qm_hbm.at[bidx_v], rows_v, sem_g)
        cp_g.start()
        cp_k.wait()
        cp_g.wait()

        def dot_body(j, _):
            acc = keys_v[j, pl.ds(0, 16)] * rows_v[j, pl.ds(0, 16)]
            for c in range(1, 16):
                acc = acc + keys_v[j, pl.ds(c * 16, 16)] * rows_v[j, pl.ds(c * 16, 16)]
            sc_v[k0 + j] = jnp.sum(acc)
            return 0

        lax.fori_loop(0, BLK, dot_body, 0, unroll=2)

        for v in range(BLK // L):
            kl = k0 + v * L
            svec = sc_v[pl.ds(kl, L)] * jnp.float32(INV_SQRT_D)
            sc_v[pl.ds(kl, L)] = svec
            evec = jnp.exp(svec)
            e_v[pl.ds(kl, L)] = evec
            c_incl = plsc.cumsum(evec) + cv
            c_excl = c_incl - evec
            cv = cv + lax.broadcast(jnp.sum(evec), (L,))
            ioff = iota + (8 + kl)
            ic = plsc.load_gather(idx_v, [ioff])
            ip = plsc.load_gather(idx_v, [ioff - 1])
            inx = plsc.load_gather(idx_v, [ioff + 1])
            r = lax.shift_right_logical(ic, 4)
            ccol = lax.bitwise_and(ic, 15)
            plsc.store_scatter(s_acc, [r, ccol], c_excl, mask=ic != ip)
            plsc.store_scatter(e_acc, [r, ccol], c_incl, mask=ic != inx)
        return cv

    lax.fori_loop(0, nblk, block_body, jnp.zeros((L,), jnp.float32))

    def pbody(i, _):
        pz_v[i, :] = e_acc[i, :] - s_acc[i, :]
        return 0

    lax.fori_loop(0, MROWSP, pbody, 0)

    for t in range(5):
        pltpu.sync_copy(pz_v.at[pl.ds(t * 128, 128)],
                        shared.at[midx_v.at[t]], add=True)

    plsc.subcore_barrier()

    @pl.when(sid == 0)
    def _():
        pltpu.sync_copy(shared, part_hbm.at[cid])

    pltpu.sync_copy(sc_v.at[pl.ds(0, BASEK)], scores_hbm.at[pl.ds(kstart, BASEK)])
    pltpu.sync_copy(e_v.at[pl.ds(0, BASEK)], e_hbm.at[pl.ds(kstart, BASEK)])

    @pl.when(nblk == MAXB)
    def _():
        pltpu.sync_copy(sc_v.at[pl.ds(BASEK, BLK)],
                        scores_hbm.at[pl.ds(kstart + BASEK, BLK)])
        pltpu.sync_copy(e_v.at[pl.ds(BASEK, BLK)],
                        e_hbm.at[pl.ds(kstart + BASEK, BLK)])


def _pass2_body(e_hbm, idx_hbm, part_hbm, w_hbm,
                idx_v, e_v, w_v, d0_v, d1_v):
    cid = lax.axis_index("c")
    sid = lax.axis_index("s")
    wid = cid * NS + sid
    nblk, kstart = _worker_range(wid)

    pltpu.sync_copy(idx_hbm.at[pl.ds(kstart, BASEK)], idx_v.at[pl.ds(0, BASEK)])
    pltpu.sync_copy(e_hbm.at[pl.ds(kstart, BASEK)], e_v.at[pl.ds(0, BASEK)])

    @pl.when(nblk == MAXB)
    def _():
        pltpu.sync_copy(idx_hbm.at[pl.ds(kstart + BASEK, BLK)],
                        idx_v.at[pl.ds(BASEK, BLK)])
        pltpu.sync_copy(e_hbm.at[pl.ds(kstart + BASEK, BLK)],
                        e_v.at[pl.ds(BASEK, BLK)])

    pltpu.sync_copy(part_hbm.at[0], d0_v)
    pltpu.sync_copy(part_hbm.at[1], d1_v)

    def dbody(i, _):
        d0_v[i, :] = d0_v[i, :] + d1_v[i, :]
        return 0

    lax.fori_loop(0, MROWSP, dbody, 0)

    def vbody(vv, _):
        kl = vv * L
        ic = idx_v[pl.ds(kl, L)]
        r = lax.shift_right_logical(ic, 4)
        ccol = lax.bitwise_and(ic, 15)
        d = plsc.load_gather(d0_v, [r, ccol])
        w_v[pl.ds(kl, L)] = e_v[pl.ds(kl, L)] / d
        return 0

    lax.fori_loop(0, nblk * (BLK // L), vbody, 0)

    pltpu.sync_copy(w_v.at[pl.ds(0, BASEK)], w_hbm.at[pl.ds(kstart, BASEK)])

    @pl.when(nblk == MAXB)
    def _():
        pltpu.sync_copy(w_v.at[pl.ds(BASEK, BLK)],
                        w_hbm.at[pl.ds(kstart + BASEK, BLK)])


def _sc_mesh():
    return plsc.VectorSubcoreMesh(core_axis_name="c", subcore_axis_name="s")


def _pass1(qm, key_batch, idx):
    return pl.kernel(
        _pass1_body,
        out_type=[
            jax.ShapeDtypeStruct((N_KEYS,), jnp.float32),
            jax.ShapeDtypeStruct((N_KEYS,), jnp.float32),
            jax.ShapeDtypeStruct((NC, MROWSP, L), jnp.float32),
        ],
        mesh=_sc_mesh(),
        scratch_types=[
            pltpu.VMEM((8 + MAXK + 8,), jnp.int32),
            pltpu.VMEM((BLK, D), jnp.float32),
            pltpu.VMEM((BLK, D), jnp.float32),
            pltpu.VMEM((BLK,), jnp.int32),
            pltpu.VMEM((MAXK,), jnp.float32),
            pltpu.VMEM((MAXK,), jnp.float32),
            pltpu.VMEM((MROWSP, L), jnp.float32),
            pltpu.VMEM((MROWSP, L), jnp.float32),
            pltpu.VMEM((MROWSP, L), jnp.float32),
            pltpu.VMEM((5, 128), jnp.int32),
            pltpu.VMEM_SHARED((MROWSP, L), jnp.float32),
            pltpu.SemaphoreType.DMA,
            pltpu.SemaphoreType.DMA,
        ],
    )(qm, key_batch, idx)


def _pass2(e, idx, part):
    return pl.kernel(
        _pass2_body,
        out_type=jax.ShapeDtypeStruct((N_KEYS,), jnp.float32),
        mesh=_sc_mesh(),
        scratch_types=[
            pltpu.VMEM((MAXK,), jnp.int32),
            pltpu.VMEM((MAXK,), jnp.float32),
            pltpu.VMEM((MAXK,), jnp.float32),
            pltpu.VMEM((MROWSP, L), jnp.float32),
            pltpu.VMEM((MROWSP, L), jnp.float32),
        ],
    )(e, idx, part)


def kernel(query_batch, key_batch, original_mol_idx_for_keys, WQ, WK):
    qm = _compute_qm(query_batch, WQ, WK)
    scores, e, part = _pass1(qm, key_batch, original_mol_idx_for_keys)
    weights = _pass2(e, original_mol_idx_for_keys, part)
    return (weights, scores)


# speedup vs baseline: 7.4134x; 7.4134x over previous
import jax
import jax.numpy as jnp
from jax import lax
from jax.experimental import pallas as pl
from jax.experimental.pallas import tpu as pltpu
from jax.experimental.pallas import tpu_sc as plsc

N_MOLS = 10000
N_KEYS = 160000
D = 256
L = 16
NC = 2
NS = 16
NW = NC * NS
BLK = 128
NBLK_TOTAL = N_KEYS // BLK
BASE_BLKS = NBLK_TOTAL // NW
EXTRA = NBLK_TOTAL - BASE_BLKS * NW
MAXB = BASE_BLKS + 1
MAXK = MAXB * BLK
BASEK = BASE_BLKS * BLK
NMP = 10240
MR = 80
INV_SQRT_D = 0.0625


def _qm_body(q_ref, wq_ref, wk_ref, out_ref):
    m = lax.dot_general(
        wq_ref[...], wk_ref[...], (((0,), (0,)), ((), ())),
        preferred_element_type=jnp.float32, precision=lax.Precision.HIGHEST)
    out_ref[...] = lax.dot_general(
        q_ref[...], m, (((1,), (0,)), ((), ())),
        preferred_element_type=jnp.float32, precision=lax.Precision.HIGHEST)


def _compute_qm(query, wq, wk):
    GB = 1000
    return pl.pallas_call(
        _qm_body,
        grid=(N_MOLS // GB,),
        in_specs=[
            pl.BlockSpec((GB, D), lambda i: (i, 0)),
            pl.BlockSpec((D, D), lambda i: (0, 0)),
            pl.BlockSpec((D, D), lambda i: (0, 0)),
        ],
        out_specs=pl.BlockSpec((GB, D), lambda i: (i, 0)),
        out_shape=jax.ShapeDtypeStruct((N_MOLS, D), jnp.float32),
    )(query, wq, wk)


def _worker_range(wid):
    nblk = BASE_BLKS + jnp.where(wid < EXTRA, 1, 0)
    kstart = (wid * BASE_BLKS + jnp.minimum(wid, EXTRA)) * BLK
    return nblk, kstart


def _pass1_body(qm_hbm, key_hbm, idx_hbm, scores_hbm, e_hbm, part_hbm,
                idx_v, keys_v, rows_v, bidx_v, sc_v, e_v, s_acc, e_acc,
                pz_v, midx_v, shared, sem_k, sem_g):
    cid = lax.axis_index("c")
    sid = lax.axis_index("s")
    wid = cid * NS + sid
    nblk, kstart = _worker_range(wid)
    nkeys = nblk * BLK

    zer = jnp.zeros((L,), jnp.float32)

    def zbody(i, _):
        f = i * L
        s_acc[pl.ds(f, L)] = zer
        e_acc[pl.ds(f, L)] = zer
        pz_v[lax.shift_right_logical(i, 3),
             pl.ds(lax.bitwise_and(i, 7) * L, L)] = zer
        return 0

    lax.fori_loop(0, NMP // L, zbody, 0)

    iota = lax.iota(jnp.int32, L)
    for t in range(MR // 16):
        midx_v[pl.ds(t * 16, 16)] = iota + t * 16

    @pl.when(sid == 0)
    def _():
        pltpu.sync_copy(pz_v, shared)

    plsc.subcore_barrier()

    idx_v[pl.ds(0, 16)] = jnp.full((L,), -1, jnp.int32)
    pltpu.sync_copy(idx_hbm.at[pl.ds(kstart, BASEK)], idx_v.at[pl.ds(8, BASEK)])

    @pl.when(nblk == MAXB)
    def _():
        pltpu.sync_copy(idx_hbm.at[pl.ds(kstart + BASEK, BLK)],
                        idx_v.at[pl.ds(8 + BASEK, BLK)])

    idx_v[pl.ds(8 + nkeys, 16)] = jnp.full((L,), -2, jnp.int32)

    def block_body(b, cv):
        k0 = b * BLK
        for j8 in range(BLK // 16):
            bidx_v[pl.ds(j8 * 16, 16)] = idx_v[pl.ds(8 + k0 + j8 * 16, 16)]
        cp_k = pltpu.make_async_copy(
            key_hbm.at[pl.ds(kstart + k0, BLK)], keys_v, sem_k)
        cp_k.start()
        cp_g = pltpu.make_async_copy(qm_hbm.at[bidx_v], rows_v, sem_g)
        cp_g.start()
        cp_k.wait()
        cp_g.wait()

        for v in range(BLK // L):
            kl = k0 + v * L

            def key_body(j, sv):
                jj = v * L + j
                acc = keys_v[jj, pl.ds(0, 16)] * rows_v[jj, pl.ds(0, 16)]
                for c in range(1, 16):
                    acc = acc + (keys_v[jj, pl.ds(c * 16, 16)]
                                 * rows_v[jj, pl.ds(c * 16, 16)])
                return jnp.where(iota == j,
                                 lax.broadcast(jnp.sum(acc), (L,)), sv)

            sraw = lax.fori_loop(0, L, key_body, zer)
            svec = sraw * jnp.float32(INV_SQRT_D)
            sc_v[pl.ds(kl, L)] = svec
            evec = jnp.exp(svec)
            e_v[pl.ds(kl, L)] = evec
            c_incl = plsc.cumsum(evec) + cv
            c_excl = c_incl - evec
            cv = cv + lax.broadcast(jnp.sum(evec), (L,))
            ioff = iota + (8 + kl)
            ic = plsc.load_gather(idx_v, [ioff])
            ip = plsc.load_gather(idx_v, [ioff - 1])
            inx = plsc.load_gather(idx_v, [ioff + 1])
            plsc.store_scatter(s_acc, [ic], c_excl, mask=ic != ip)
            plsc.store_scatter(e_acc, [ic], c_incl, mask=ic != inx)
        return cv

    lax.fori_loop(0, nblk, block_body, jnp.zeros((L,), jnp.float32))

    def pbody(i, _):
        f = i * L
        pz_v[lax.shift_right_logical(i, 3),
             pl.ds(lax.bitwise_and(i, 7) * L, L)] = (
            e_acc[pl.ds(f, L)] - s_acc[pl.ds(f, L)])
        return 0

    lax.fori_loop(0, NMP // L, pbody, 0)

    pltpu.sync_copy(pz_v, shared.at[midx_v], add=True)

    plsc.subcore_barrier()

    @pl.when(sid == 0)
    def _():
        pltpu.sync_copy(shared, part_hbm.at[cid])

    pltpu.sync_copy(sc_v.at[pl.ds(0, BASEK)], scores_hbm.at[pl.ds(kstart, BASEK)])
    pltpu.sync_copy(e_v.at[pl.ds(0, BASEK)], e_hbm.at[pl.ds(kstart, BASEK)])

    @pl.when(nblk == MAXB)
    def _():
        pltpu.sync_copy(sc_v.at[pl.ds(BASEK, BLK)],
                        scores_hbm.at[pl.ds(kstart + BASEK, BLK)])
        pltpu.sync_copy(e_v.at[pl.ds(BASEK, BLK)],
                        e_hbm.at[pl.ds(kstart + BASEK, BLK)])


def _pass2_body(e_hbm, idx_hbm, part_hbm, w_hbm,
                idx_v, e_v, w_v, d0_v, d1_v):
    cid = lax.axis_index("c")
    sid = lax.axis_index("s")
    wid = cid * NS + sid
    nblk, kstart = _worker_range(wid)

    pltpu.sync_copy(idx_hbm.at[pl.ds(kstart, BASEK)], idx_v.at[pl.ds(0, BASEK)])
    pltpu.sync_copy(e_hbm.at[pl.ds(kstart, BASEK)], e_v.at[pl.ds(0, BASEK)])

    @pl.when(nblk == MAXB)
    def _():
        pltpu.sync_copy(idx_hbm.at[pl.ds(kstart + BASEK, BLK)],
                        idx_v.at[pl.ds(BASEK, BLK)])
        pltpu.sync_copy(e_hbm.at[pl.ds(kstart + BASEK, BLK)],
                        e_v.at[pl.ds(BASEK, BLK)])

    pltpu.sync_copy(part_hbm.at[0], d0_v)
    pltpu.sync_copy(part_hbm.at[1], d1_v)

    def dbody(i, _):
        r = lax.shift_right_logical(i, 3)
        c = lax.bitwise_and(i, 7) * L
        d0_v[r, pl.ds(c, L)] = d0_v[r, pl.ds(c, L)] + d1_v[r, pl.ds(c, L)]
        return 0

    lax.fori_loop(0, NMP // L, dbody, 0)

    def vbody(vv, _):
        kl = vv * L
        ic = idx_v[pl.ds(kl, L)]
        r = lax.shift_right_logical(ic, 7)
        ccol = lax.bitwise_and(ic, 127)
        d = plsc.load_gather(d0_v, [r, ccol])
        w_v[pl.ds(kl, L)] = e_v[pl.ds(kl, L)] / d
        return 0

    lax.fori_loop(0, nblk * (BLK // L), vbody, 0)

    pltpu.sync_copy(w_v.at[pl.ds(0, BASEK)], w_hbm.at[pl.ds(kstart, BASEK)])

    @pl.when(nblk == MAXB)
    def _():
        pltpu.sync_copy(w_v.at[pl.ds(BASEK, BLK)],
                        w_hbm.at[pl.ds(kstart + BASEK, BLK)])


def _sc_mesh():
    return plsc.VectorSubcoreMesh(core_axis_name="c", subcore_axis_name="s")


def _pass1(qm, key_batch, idx):
    return pl.kernel(
        _pass1_body,
        out_type=[
            jax.ShapeDtypeStruct((N_KEYS,), jnp.float32),
            jax.ShapeDtypeStruct((N_KEYS,), jnp.float32),
            jax.ShapeDtypeStruct((NC, MR, 128), jnp.float32),
        ],
        mesh=_sc_mesh(),
        compiler_params=pltpu.CompilerParams(needs_layout_passes=False),
        scratch_types=[
            pltpu.VMEM((8 + MAXK + 24,), jnp.int32),
            pltpu.VMEM((BLK, D), jnp.float32),
            pltpu.VMEM((BLK, D), jnp.float32),
            pltpu.VMEM((BLK,), jnp.int32),
            pltpu.VMEM((MAXK,), jnp.float32),
            pltpu.VMEM((MAXK,), jnp.float32),
            pltpu.VMEM((NMP,), jnp.float32),
            pltpu.VMEM((NMP,), jnp.float32),
            pltpu.VMEM((MR, 128), jnp.float32),
            pltpu.VMEM((MR,), jnp.int32),
            pltpu.VMEM_SHARED((MR, 128), jnp.float32),
            pltpu.SemaphoreType.DMA,
            pltpu.SemaphoreType.DMA,
        ],
    )(qm, key_batch, idx)


def _pass2(e, idx, part):
    return pl.kernel(
        _pass2_body,
        out_type=jax.ShapeDtypeStruct((N_KEYS,), jnp.float32),
        mesh=_sc_mesh(),
        compiler_params=pltpu.CompilerParams(needs_layout_passes=False),
        scratch_types=[
            pltpu.VMEM((MAXK,), jnp.int32),
            pltpu.VMEM((MAXK,), jnp.float32),
            pltpu.VMEM((MAXK,), jnp.float32),
            pltpu.VMEM((MR, 128), jnp.float32),
            pltpu.VMEM((MR, 128), jnp.float32),
        ],
    )(e, idx, part)


def kernel(query_batch, key_batch, original_mol_idx_for_keys, WQ, WK):
    qm = _compute_qm(query_batch, WQ, WK)
    scores, e, part = _pass1(qm, key_batch, original_mol_idx_for_keys)
    weights = _pass2(e, original_mol_idx_for_keys, part)
    return (weights, scores)
